# Initial kernel scaffold; baseline (speedup 1.0000x reference)
import jax
import jax.numpy as jnp
from jax.experimental import pallas as pl


def kernel(atom_fea, nbr_fea, nbr_fea_idx, crystal_atom_idx, params):
    raise NotImplementedError("write your pallas kernel here")


# speedup vs baseline: 1.1516x; 1.1516x over previous
import functools

import jax
import jax.numpy as jnp
from jax import lax
from jax.experimental import pallas as pl
from jax.experimental.pallas import tpu as pltpu
from jax.experimental.pallas import tpu_sc as plsc

N = 10000
M = 32
E = N * M
ORIG = 128
NBR = 16
AFL = 64
NC = 128
NA = 78

SC_CORES = 2
SC_SUB = 16
NW = SC_CORES * SC_SUB
EW = E // NW
IDXW = 80
NROW = EW // IDXW
GRP = 5
NOUT = NROW // GRP
CH = IDXW * GRP
NP = 10240
RPS = NP // SC_SUB

PIDXW = 104
PROWS = 3
PW = PIDXW * PROWS


def _mm(a, b):
    return jnp.dot(a, b, precision=lax.Precision.HIGHEST)


def _embed_body(a_ref, w_ref, b_ref, o_ref):
    o_ref[...] = _mm(a_ref[...], w_ref[...]) + b_ref[...]


def _embed(atom_fea, w, b):
    return pl.pallas_call(
        _embed_body,
        grid=(10,),
        in_specs=[
            pl.BlockSpec((N // 10, ORIG), lambda i: (i, 0)),
            pl.BlockSpec((ORIG, AFL), lambda i: (0, 0)),
            pl.BlockSpec((1, AFL), lambda i: (0, 0)),
        ],
        out_specs=pl.BlockSpec((N // 10, AFL), lambda i: (i, 0)),
        out_shape=jax.ShapeDtypeStruct((N, AFL), jnp.float32),
    )(atom_fea, w, b.reshape(1, AFL))


_BE = 1280
_GA = _BE // M


def _edge_mlp_body(nbr_ref, w1, b1, w2, b2, w3, b3, t_ref, ezs_ref):
    h = jnp.maximum(_mm(nbr_ref[...], w1[...]) + b1[...], 0.0)
    h = jnp.maximum(_mm(h, w2[...]) + b2[...], 0.0)
    t = _mm(h, w3[...]) + b3[...]
    t_ref[...] = t
    ezs_ref[...] = jnp.sum(t.reshape(_GA, M, AFL), axis=1)


def _edge_mlp(nbr_flat, w1, b1, w2, b2, w3, b3):
    return pl.pallas_call(
        _edge_mlp_body,
        grid=(E // _BE,),
        in_specs=[
            pl.BlockSpec((_BE, NBR), lambda i: (i, 0)),
            pl.BlockSpec((NBR, 256), lambda i: (0, 0)),
            pl.BlockSpec((1, 256), lambda i: (0, 0)),
            pl.BlockSpec((256, 128), lambda i: (0, 0)),
            pl.BlockSpec((1, 128), lambda i: (0, 0)),
            pl.BlockSpec((128, AFL), lambda i: (0, 0)),
            pl.BlockSpec((1, AFL), lambda i: (0, 0)),
        ],
        out_specs=[
            pl.BlockSpec((_BE, AFL), lambda i: (i, 0)),
            pl.BlockSpec((_GA, AFL), lambda i: (i, 0)),
        ],
        out_shape=[
            jax.ShapeDtypeStruct((E, AFL), jnp.float32),
            jax.ShapeDtypeStruct((N, AFL), jnp.float32),
        ],
    )(nbr_flat, w1, b1.reshape(1, 256), w2, b2.reshape(1, 128), w3,
      b3.reshape(1, AFL))


def _bn_body(a_ref, x_ref):
    a = a_ref[...]
    m = jnp.mean(a, axis=0, keepdims=True)
    v = jnp.mean((a - m) * (a - m), axis=0, keepdims=True)
    x_ref[...] = (a - m) * lax.rsqrt(v + 1e-5)


def _bn(a):
    return pl.pallas_call(
        _bn_body,
        out_shape=jax.ShapeDtypeStruct((N, AFL), jnp.float32),
    )(a)


def _comb_body(x_ref, ezs_ref, ezt_ref, o_ref):
    ezt = ezt_ref[...]
    o_ref[...] = x_ref[...] + ezs_ref[...] + ezt[0, :N] + ezt[1, :N]


def _combine(x, ezs, ezt_p):
    return pl.pallas_call(
        _comb_body,
        out_shape=jax.ShapeDtypeStruct((N, AFL), jnp.float32),
    )(x, ezs, ezt_p)


def _msg_body(xi_ref, comb_ref, wa, wb, be1, we2, be2, o_ref):
    cb = _mm(comb_ref[...], wb[...]) + be1[...]
    cbr = jnp.broadcast_to(cb.reshape(_GA, 1, 256),
                           (_GA, M, 256)).reshape(_BE, 256)
    h = jnp.maximum(_mm(xi_ref[...], wa[...]) + cbr, 0.0)
    o_ref[...] = _mm(h, we2[...]) + be2[...]


def _msg(xi, comb, wa, wb, be1, we2, be2):
    return pl.pallas_call(
        _msg_body,
        grid=(E // _BE,),
        in_specs=[
            pl.BlockSpec((_BE, AFL), lambda i: (i, 0)),
            pl.BlockSpec((_GA, AFL), lambda i: (i, 0)),
            pl.BlockSpec((AFL, 256), lambda i: (0, 0)),
            pl.BlockSpec((AFL, 256), lambda i: (0, 0)),
            pl.BlockSpec((1, 256), lambda i: (0, 0)),
            pl.BlockSpec((256, AFL), lambda i: (0, 0)),
            pl.BlockSpec((1, AFL), lambda i: (0, 0)),
        ],
        out_specs=pl.BlockSpec((_BE, AFL), lambda i: (i, 0)),
        out_shape=jax.ShapeDtypeStruct((E, AFL), jnp.float32),
    )(xi, comb, wa, wb, be1.reshape(1, 256), we2, be2.reshape(1, AFL))


def _tail_body(s_ref, cnt_ref, comb_ref, o_ref):
    cnt_p = cnt_ref[...]
    cnt = jnp.maximum(cnt_p[0, :N, :1] + cnt_p[1, :N, :1], 1.0)
    s_p = s_ref[...]
    agg = (s_p[0, :N] + s_p[1, :N]) / cnt
    m = jnp.mean(agg, axis=0, keepdims=True)
    v = jnp.mean((agg - m) * (agg - m), axis=0, keepdims=True)
    o_ref[...] = (agg - m) * lax.rsqrt(v + 1e-5) + comb_ref[...]


def _tail(s_p, cnt_p, comb):
    return pl.pallas_call(
        _tail_body,
        out_shape=jax.ShapeDtypeStruct((N, AFL), jnp.float32),
    )(s_p, cnt_p, comb)


def _head_body(g_ref, wfc, bfc, wh1, bh1, wh2, bh2, wo, bo, o_ref):
    pooled = jnp.mean(g_ref[...], axis=1)
    h = _mm(pooled, wfc[...]) + bfc[...]
    h = jnp.maximum(_mm(h, wh1[...]) + bh1[...], 0.0)
    h = jnp.maximum(_mm(h, wh2[...]) + bh2[...], 0.0)
    o_ref[...] = _mm(h, wo[...]) + bo[...]


def _head(gathered, p):
    return pl.pallas_call(
        _head_body,
        out_shape=jax.ShapeDtypeStruct((NC, 1), jnp.float32),
    )(gathered.reshape(NC, NA, AFL), p['W_fc'], p['b_fc'].reshape(1, 256),
      p['Wh1'], p['bh1'].reshape(1, 128), p['Wh2'], p['bh2'].reshape(1, 64),
      p['Wout'], p['bout'].reshape(1, 1))


@functools.cache
def _mesh():
    return plsc.VectorSubcoreMesh(core_axis_name="c", subcore_axis_name="s",
                                  num_cores=SC_CORES, num_subcores=SC_SUB)


def _sc_scatter_gather_cnt_body(t_hbm, x_hbm, idx_hbm, z64, z16, ones_hbm,
                                ezt_out, xi_out, cnt_out,
                                idx_v, tbuf, xbuf, ones_v, sem,
                                acc64, acc16):
    c = lax.axis_index("c")
    s = lax.axis_index("s")
    w = c * SC_SUB + s
    pltpu.sync_copy(idx_hbm.at[w], idx_v)
    pltpu.sync_copy(ones_hbm, ones_v)
    pltpu.sync_copy(z64.at[pl.ds(pl.multiple_of(s * RPS, 8), RPS)], acc64.at[pl.ds(pl.multiple_of(s * RPS, 8), RPS)])
    pltpu.sync_copy(z16.at[pl.ds(pl.multiple_of(s * RPS, 8), RPS)], acc16.at[pl.ds(pl.multiple_of(s * RPS, 8), RPS)])
    plsc.subcore_barrier()

    def outer(j, carry):
        base = pl.multiple_of(w * EW + j * CH, 8)
        pltpu.sync_copy(t_hbm.at[pl.ds(base, CH)], tbuf)
        for i in range(GRP):
            row = j * GRP + i
            pltpu.sync_copy(tbuf.at[pl.ds(i * IDXW, IDXW)],
                            acc64.at[idx_v.at[row]], add=True)
            pltpu.sync_copy(ones_v, acc16.at[idx_v.at[row]], add=True)
            pltpu.async_copy(x_hbm.at[idx_v.at[row]],
                             xbuf.at[pl.ds(i * IDXW, IDXW)], sem).wait()
        pltpu.sync_copy(xbuf, xi_out.at[pl.ds(base, CH)])
        return carry

    lax.fori_loop(0, NOUT, outer, 0)
    plsc.subcore_barrier()
    pltpu.sync_copy(acc64.at[pl.ds(pl.multiple_of(s * RPS, 8), RPS)],
                    ezt_out.at[c, pl.ds(pl.multiple_of(s * RPS, 8), RPS)])
    pltpu.sync_copy(acc16.at[pl.ds(pl.multiple_of(s * RPS, 8), RPS)],
                    cnt_out.at[c, pl.ds(pl.multiple_of(s * RPS, 8), RPS)])


def _sc_scatter_gather_body(t_hbm, x_hbm, idx_hbm, z64,
                            ezt_out, xi_out,
                            idx_v, tbuf, xbuf, sem, acc64):
    c = lax.axis_index("c")
    s = lax.axis_index("s")
    w = c * SC_SUB + s
    pltpu.sync_copy(idx_hbm.at[w], idx_v)
    pltpu.sync_copy(z64.at[pl.ds(pl.multiple_of(s * RPS, 8), RPS)], acc64.at[pl.ds(pl.multiple_of(s * RPS, 8), RPS)])
    plsc.subcore_barrier()

    def outer(j, carry):
        base = pl.multiple_of(w * EW + j * CH, 8)
        pltpu.sync_copy(t_hbm.at[pl.ds(base, CH)], tbuf)
        for i in range(GRP):
            row = j * GRP + i
            pltpu.sync_copy(tbuf.at[pl.ds(i * IDXW, IDXW)],
                            acc64.at[idx_v.at[row]], add=True)
            pltpu.async_copy(x_hbm.at[idx_v.at[row]],
                             xbuf.at[pl.ds(i * IDXW, IDXW)], sem).wait()
        pltpu.sync_copy(xbuf, xi_out.at[pl.ds(base, CH)])
        return carry

    lax.fori_loop(0, NOUT, outer, 0)
    plsc.subcore_barrier()
    pltpu.sync_copy(acc64.at[pl.ds(pl.multiple_of(s * RPS, 8), RPS)],
                    ezt_out.at[c, pl.ds(pl.multiple_of(s * RPS, 8), RPS)])


def _sc_scatter_body(v_hbm, idx_hbm, z64, s_out, idx_v, vbuf, acc64):
    c = lax.axis_index("c")
    s = lax.axis_index("s")
    w = c * SC_SUB + s
    pltpu.sync_copy(idx_hbm.at[w], idx_v)
    pltpu.sync_copy(z64.at[pl.ds(pl.multiple_of(s * RPS, 8), RPS)], acc64.at[pl.ds(pl.multiple_of(s * RPS, 8), RPS)])
    plsc.subcore_barrier()

    def outer(j, carry):
        base = pl.multiple_of(w * EW + j * CH, 8)
        pltpu.sync_copy(v_hbm.at[pl.ds(base, CH)], vbuf)
        for i in range(GRP):
            row = j * GRP + i
            pltpu.sync_copy(vbuf.at[pl.ds(i * IDXW, IDXW)],
                            acc64.at[idx_v.at[row]], add=True)
        return carry

    lax.fori_loop(0, NOUT, outer, 0)
    plsc.subcore_barrier()
    pltpu.sync_copy(acc64.at[pl.ds(pl.multiple_of(s * RPS, 8), RPS)],
                    s_out.at[c, pl.ds(pl.multiple_of(s * RPS, 8), RPS)])


def _sc_pool_body(af_hbm, cidx_hbm, g_out, cidx_v, gbuf, sem):
    c = lax.axis_index("c")
    s = lax.axis_index("s")
    w = c * SC_SUB + s
    pltpu.sync_copy(cidx_hbm.at[w], cidx_v)
    for i in range(PROWS):
        pltpu.async_copy(af_hbm.at[cidx_v.at[i]],
                         gbuf.at[pl.ds(i * PIDXW, PIDXW)], sem).wait()
    pltpu.sync_copy(gbuf, g_out.at[pl.ds(pl.multiple_of(w * PW, 8), PW)])


@functools.cache
def _get_sc_scatter_gather_cnt():
  return pl.kernel(
    _sc_scatter_gather_cnt_body,
    out_type=(
        jax.ShapeDtypeStruct((SC_CORES, NP, AFL), jnp.float32),
        jax.ShapeDtypeStruct((E, AFL), jnp.float32),
        jax.ShapeDtypeStruct((SC_CORES, NP, 16), jnp.float32),
    ),
    mesh=_mesh(),
    compiler_params=pltpu.CompilerParams(use_tc_tiling_on_sc=False),
    scratch_types=[
        pltpu.VMEM((NROW, IDXW), jnp.int32),
        pltpu.VMEM((CH, AFL), jnp.float32),
        pltpu.VMEM((CH, AFL), jnp.float32),
        pltpu.VMEM((IDXW, 16), jnp.float32),
        pltpu.SemaphoreType.DMA,
        pltpu.VMEM_SHARED((NP, AFL), jnp.float32),
        pltpu.VMEM_SHARED((NP, 16), jnp.float32),
    ],
)

@functools.cache
def _get_sc_scatter_gather():
  return pl.kernel(
    _sc_scatter_gather_body,
    out_type=(
        jax.ShapeDtypeStruct((SC_CORES, NP, AFL), jnp.float32),
        jax.ShapeDtypeStruct((E, AFL), jnp.float32),
    ),
    mesh=_mesh(),
    compiler_params=pltpu.CompilerParams(use_tc_tiling_on_sc=False),
    scratch_types=[
        pltpu.VMEM((NROW, IDXW), jnp.int32),
        pltpu.VMEM((CH, AFL), jnp.float32),
        pltpu.VMEM((CH, AFL), jnp.float32),
        pltpu.SemaphoreType.DMA,
        pltpu.VMEM_SHARED((NP, AFL), jnp.float32),
    ],
)

@functools.cache
def _get_sc_scatter():
  return pl.kernel(
    _sc_scatter_body,
    out_type=jax.ShapeDtypeStruct((SC_CORES, NP, AFL), jnp.float32),
    mesh=_mesh(),
    compiler_params=pltpu.CompilerParams(use_tc_tiling_on_sc=False),
    scratch_types=[
        pltpu.VMEM((NROW, IDXW), jnp.int32),
        pltpu.VMEM((CH, AFL), jnp.float32),
        pltpu.VMEM_SHARED((NP, AFL), jnp.float32),
    ],
)

@functools.cache
def _get_sc_pool():
  return pl.kernel(
    _sc_pool_body,
    out_type=jax.ShapeDtypeStruct((NC * NA, AFL), jnp.float32),
    mesh=_mesh(),
    compiler_params=pltpu.CompilerParams(use_tc_tiling_on_sc=False),
    scratch_types=[
        pltpu.VMEM((PROWS, PIDXW), jnp.int32),
        pltpu.VMEM((PW, AFL), jnp.float32),
        pltpu.SemaphoreType.DMA,
    ],
)


def _sc_scatter_gather_cnt(*args):
    return _get_sc_scatter_gather_cnt()(*args)


def _sc_scatter_gather(*args):
    return _get_sc_scatter_gather()(*args)


def _sc_scatter(*args):
    return _get_sc_scatter()(*args)


def _sc_pool(*args):
    return _get_sc_pool()(*args)


def kernel(atom_fea, nbr_fea, nbr_fea_idx, crystal_atom_idx, params):
    p = params
    tgt3d = nbr_fea_idx.astype(jnp.int32).reshape(NW, NROW, IDXW)
    cidx3d = crystal_atom_idx.astype(jnp.int32).reshape(NW, PROWS, PIDXW)
    z64 = jnp.zeros((NP, AFL), jnp.float32)
    z16 = jnp.zeros((NP, 16), jnp.float32)
    ones16 = jnp.ones((IDXW, 16), jnp.float32)
    nbr_flat = nbr_fea.reshape(E, NBR)

    af = _embed(atom_fea, p['W_emb'], p['b_emb'])
    cnt_p = None
    for l in range(3):
        t, ezs = _edge_mlp(nbr_flat,
                           p['c%d_W1' % l], p['c%d_b1' % l],
                           p['c%d_W2' % l], p['c%d_b2' % l],
                           p['c%d_W3' % l], p['c%d_b3' % l])
        x = _bn(af)
        if l == 0:
            ezt_p, xi, cnt_p = _sc_scatter_gather_cnt(
                t, x, tgt3d, z64, z16, ones16)
        else:
            ezt_p, xi = _sc_scatter_gather(t, x, tgt3d, z64)
        comb = _combine(x, ezs, ezt_p)
        we1 = p['c%d_We1' % l]
        wa = we1[:AFL] - we1[AFL:]
        wb = we1[AFL:]
        msg = _msg(xi, comb, wa, wb, p['c%d_be1' % l],
                   p['c%d_We2' % l], p['c%d_be2' % l])
        s_p = _sc_scatter(msg, tgt3d, z64)
        af = _tail(s_p, cnt_p, comb)

    gathered = _sc_pool(af, cidx3d)
    return _head(gathered, p)


# speedup vs baseline: 2.4820x; 2.1552x over previous
import functools

import jax
import jax.numpy as jnp
from jax import lax
from jax.experimental import pallas as pl
from jax.experimental.pallas import tpu as pltpu
from jax.experimental.pallas import tpu_sc as plsc

N = 10000
M = 32
E = N * M
ORIG = 128
NBR = 16
AFL = 64
NC = 128
NA = 78

SC_CORES = 2
SC_SUB = 16
NW = SC_CORES * SC_SUB
EW = E // NW
IDXW = 80
NROW = EW // IDXW
GRP = 5
NOUT = NROW // GRP
CH = IDXW * GRP
NP = 10240
RPS = NP // SC_SUB

PIDXW = 104
PROWS = 3
PW = PIDXW * PROWS


def _mm(a, b):
    return _dot(a.astype(jnp.bfloat16), b.astype(jnp.bfloat16))


def _dot(a, b):
    return lax.dot_general(a, b, (((1,), (0,)), ((), ())),
                           preferred_element_type=jnp.float32)


def _mm3(a, bh, bl):
    del bl
    return _dot(a.astype(jnp.bfloat16), bh)


def _split(w):
    hi = w.astype(jnp.bfloat16)
    lo = (w - hi.astype(jnp.float32)).astype(jnp.bfloat16)
    return hi, lo


def _blkdiag(w, k):
    kk, jj = w.shape
    eye = jnp.eye(k, dtype=w.dtype)
    return (eye[:, None, :, None] * w[None, :, None, :]).reshape(k * kk, k * jj)


def _embed_body(a_ref, w_ref, b_ref, o_ref):
    o_ref[...] = _mm(a_ref[...], w_ref[...]) + b_ref[...]


def _embed(atom_fea, w, b):
    return pl.pallas_call(
        _embed_body,
        grid=(10,),
        in_specs=[
            pl.BlockSpec((N // 10, ORIG), lambda i: (i, 0)),
            pl.BlockSpec((ORIG, AFL), lambda i: (0, 0)),
            pl.BlockSpec((1, AFL), lambda i: (0, 0)),
        ],
        out_specs=pl.BlockSpec((N // 10, AFL), lambda i: (i, 0)),
        out_shape=jax.ShapeDtypeStruct((N, AFL), jnp.float32),
    )(atom_fea, w, b.reshape(1, AFL))


_BE = 1280
_GA = _BE // M


def _edge_mlp_body(nbr8_ref, w1h, w1l, b1t, w2h, w2l, b2t, w3h, w3l, b3t,
                   t_ref, ezs_ref):
    h = jnp.maximum(_mm3(nbr8_ref[...], w1h[...], w1l[...]) + b1t[...], 0.0)
    h = jnp.maximum(_mm3(h, w2h[...], w2l[...]) + b2t[...], 0.0)
    tp = _mm3(h, w3h[...], w3l[...]) + b3t[...]
    t_ref[...] = tp
    ezs_ref[...] = jnp.sum(tp.reshape(_GA, M // 8, 8, AFL), axis=(1, 2))


def _edge_mlp(nbr8, w1, b1, w2, b2, w3, b3):
    tp, ezs = pl.pallas_call(
        _edge_mlp_body,
        grid=(E // _BE,),
        in_specs=[
            pl.BlockSpec((_BE // 8, 8 * NBR), lambda i: (i, 0)),
            pl.BlockSpec((8 * NBR, 8 * 256), lambda i: (0, 0)),
            pl.BlockSpec((8 * NBR, 8 * 256), lambda i: (0, 0)),
            pl.BlockSpec((1, 8 * 256), lambda i: (0, 0)),
            pl.BlockSpec((8 * 256, 8 * 128), lambda i: (0, 0)),
            pl.BlockSpec((8 * 256, 8 * 128), lambda i: (0, 0)),
            pl.BlockSpec((1, 8 * 128), lambda i: (0, 0)),
            pl.BlockSpec((8 * 128, 8 * AFL), lambda i: (0, 0)),
            pl.BlockSpec((8 * 128, 8 * AFL), lambda i: (0, 0)),
            pl.BlockSpec((1, 8 * AFL), lambda i: (0, 0)),
        ],
        out_specs=[
            pl.BlockSpec((_BE // 8, 8 * AFL), lambda i: (i, 0)),
            pl.BlockSpec((_GA, AFL), lambda i: (i, 0)),
        ],
        out_shape=[
            jax.ShapeDtypeStruct((E // 8, 8 * AFL), jnp.float32),
            jax.ShapeDtypeStruct((N, AFL), jnp.float32),
        ],
    )(nbr8, *_split(_blkdiag(w1, 8)), jnp.tile(b1, 8).reshape(1, -1),
      *_split(_blkdiag(w2, 8)), jnp.tile(b2, 8).reshape(1, -1),
      *_split(_blkdiag(w3, 8)), jnp.tile(b3, 8).reshape(1, -1))
    return tp.reshape(E, AFL), ezs


def _bn_body(a_ref, x_ref):
    a = a_ref[...]
    m = jnp.mean(a, axis=0, keepdims=True)
    v = jnp.mean((a - m) * (a - m), axis=0, keepdims=True)
    x_ref[...] = (a - m) * lax.rsqrt(v + 1e-5)


def _bn(a):
    return pl.pallas_call(
        _bn_body,
        out_shape=jax.ShapeDtypeStruct((N, AFL), jnp.float32),
    )(a)


def _comb_body(x_ref, ezs_ref, ezt_ref, o_ref):
    ezt = ezt_ref[...]
    o_ref[...] = x_ref[...] + ezs_ref[...] + ezt[0, :N] + ezt[1, :N]


def _combine(x, ezs, ezt_p):
    return pl.pallas_call(
        _comb_body,
        out_shape=jax.ShapeDtypeStruct((N, AFL), jnp.float32),
    )(x, ezs, ezt_p)


def _msg_body(xi_ref, comb_ref, we1, be1, we2, be2, o_ref):
    xi = xi_ref[...]
    xj = jnp.broadcast_to(comb_ref[...].reshape(_GA, 1, AFL),
                          (_GA, M, AFL)).reshape(_BE, AFL)
    cat = jnp.concatenate([xi, xj - xi], axis=1)
    h = jnp.maximum(_mm(cat, we1[...]) + be1[...], 0.0)
    o_ref[...] = _mm(h, we2[...]) + be2[...]


def _msg(xi, comb, we1, be1, we2, be2):
    return pl.pallas_call(
        _msg_body,
        grid=(E // _BE,),
        in_specs=[
            pl.BlockSpec((_BE, AFL), lambda i: (i, 0)),
            pl.BlockSpec((_GA, AFL), lambda i: (i, 0)),
            pl.BlockSpec((2 * AFL, 256), lambda i: (0, 0)),
            pl.BlockSpec((1, 256), lambda i: (0, 0)),
            pl.BlockSpec((256, AFL), lambda i: (0, 0)),
            pl.BlockSpec((1, AFL), lambda i: (0, 0)),
        ],
        out_specs=pl.BlockSpec((_BE, AFL), lambda i: (i, 0)),
        out_shape=jax.ShapeDtypeStruct((E, AFL), jnp.float32),
    )(xi, comb, we1, be1.reshape(1, 256), we2, be2.reshape(1, AFL))


def _tail_body(s_ref, cnt_ref, comb_ref, o_ref):
    cnt_p = cnt_ref[...]
    cnt = jnp.maximum(cnt_p[0, :N, :1] + cnt_p[1, :N, :1], 1.0)
    s_p = s_ref[...]
    agg = (s_p[0, :N] + s_p[1, :N]) / cnt
    m = jnp.mean(agg, axis=0, keepdims=True)
    v = jnp.mean((agg - m) * (agg - m), axis=0, keepdims=True)
    o_ref[...] = (agg - m) * lax.rsqrt(v + 1e-5) + comb_ref[...]


def _tail(s_p, cnt_p, comb):
    return pl.pallas_call(
        _tail_body,
        out_shape=jax.ShapeDtypeStruct((N, AFL), jnp.float32),
    )(s_p, cnt_p, comb)


def _head_body(g_ref, wfc, bfc, wh1, bh1, wh2, bh2, wo, bo, o_ref):
    pooled = jnp.mean(g_ref[...], axis=1)
    h = _mm(pooled, wfc[...]) + bfc[...]
    h = jnp.maximum(_mm(h, wh1[...]) + bh1[...], 0.0)
    h = jnp.maximum(_mm(h, wh2[...]) + bh2[...], 0.0)
    o_ref[...] = _mm(h, wo[...]) + bo[...]


def _head(gathered, p):
    return pl.pallas_call(
        _head_body,
        out_shape=jax.ShapeDtypeStruct((NC, 1), jnp.float32),
    )(gathered.reshape(NC, NA, AFL), p['W_fc'], p['b_fc'].reshape(1, 256),
      p['Wh1'], p['bh1'].reshape(1, 128), p['Wh2'], p['bh2'].reshape(1, 64),
      p['Wout'], p['bout'].reshape(1, 1))


@functools.cache
def _mesh():
    return plsc.VectorSubcoreMesh(core_axis_name="c", subcore_axis_name="s",
                                  num_cores=SC_CORES, num_subcores=SC_SUB)


def _sc_scatter_gather_cnt_body(t_hbm, x_hbm, idx_hbm, z64, z16, ones_hbm,
                                ezt_out, xi_out, cnt_out,
                                idx_v, tbuf, xbuf, ones_v, sem,
                                acc64, acc16):
    c = lax.axis_index("c")
    s = lax.axis_index("s")
    w = c * SC_SUB + s
    pltpu.sync_copy(idx_hbm.at[w], idx_v)
    pltpu.sync_copy(ones_hbm, ones_v)
    pltpu.sync_copy(z64.at[pl.ds(pl.multiple_of(s * RPS, 8), RPS)], acc64.at[pl.ds(pl.multiple_of(s * RPS, 8), RPS)])
    pltpu.sync_copy(z16.at[pl.ds(pl.multiple_of(s * RPS, 8), RPS)], acc16.at[pl.ds(pl.multiple_of(s * RPS, 8), RPS)])
    plsc.subcore_barrier()

    def outer(j, carry):
        base = pl.multiple_of(w * EW + j * CH, 8)
        pltpu.sync_copy(t_hbm.at[pl.ds(base, CH)], tbuf)
        for i in range(GRP):
            row = j * GRP + i
            pltpu.sync_copy(tbuf.at[pl.ds(i * IDXW, IDXW)],
                            acc64.at[idx_v.at[row]], add=True)
            pltpu.sync_copy(ones_v, acc16.at[idx_v.at[row]], add=True)
            pltpu.async_copy(x_hbm.at[idx_v.at[row]],
                             xbuf.at[pl.ds(i * IDXW, IDXW)], sem).wait()
        pltpu.sync_copy(xbuf, xi_out.at[pl.ds(base, CH)])
        return carry

    lax.fori_loop(0, NOUT, outer, 0)
    plsc.subcore_barrier()
    pltpu.sync_copy(acc64.at[pl.ds(pl.multiple_of(s * RPS, 8), RPS)],
                    ezt_out.at[c, pl.ds(pl.multiple_of(s * RPS, 8), RPS)])
    pltpu.sync_copy(acc16.at[pl.ds(pl.multiple_of(s * RPS, 8), RPS)],
                    cnt_out.at[c, pl.ds(pl.multiple_of(s * RPS, 8), RPS)])


def _sc_scatter_gather_body(t_hbm, x_hbm, idx_hbm, z64,
                            ezt_out, xi_out,
                            idx_v, tbuf, xbuf, sem, acc64):
    c = lax.axis_index("c")
    s = lax.axis_index("s")
    w = c * SC_SUB + s
    pltpu.sync_copy(idx_hbm.at[w], idx_v)
    pltpu.sync_copy(z64.at[pl.ds(pl.multiple_of(s * RPS, 8), RPS)], acc64.at[pl.ds(pl.multiple_of(s * RPS, 8), RPS)])
    plsc.subcore_barrier()

    def outer(j, carry):
        base = pl.multiple_of(w * EW + j * CH, 8)
        pltpu.sync_copy(t_hbm.at[pl.ds(base, CH)], tbuf)
        for i in range(GRP):
            row = j * GRP + i
            pltpu.sync_copy(tbuf.at[pl.ds(i * IDXW, IDXW)],
                            acc64.at[idx_v.at[row]], add=True)
            pltpu.async_copy(x_hbm.at[idx_v.at[row]],
                             xbuf.at[pl.ds(i * IDXW, IDXW)], sem).wait()
        pltpu.sync_copy(xbuf, xi_out.at[pl.ds(base, CH)])
        return carry

    lax.fori_loop(0, NOUT, outer, 0)
    plsc.subcore_barrier()
    pltpu.sync_copy(acc64.at[pl.ds(pl.multiple_of(s * RPS, 8), RPS)],
                    ezt_out.at[c, pl.ds(pl.multiple_of(s * RPS, 8), RPS)])


def _sc_scatter_body(v_hbm, idx_hbm, z64, s_out, idx_v, vbuf, acc64):
    c = lax.axis_index("c")
    s = lax.axis_index("s")
    w = c * SC_SUB + s
    pltpu.sync_copy(idx_hbm.at[w], idx_v)
    pltpu.sync_copy(z64.at[pl.ds(pl.multiple_of(s * RPS, 8), RPS)], acc64.at[pl.ds(pl.multiple_of(s * RPS, 8), RPS)])
    plsc.subcore_barrier()

    def outer(j, carry):
        base = pl.multiple_of(w * EW + j * CH, 8)
        pltpu.sync_copy(v_hbm.at[pl.ds(base, CH)], vbuf)
        for i in range(GRP):
            row = j * GRP + i
            pltpu.sync_copy(vbuf.at[pl.ds(i * IDXW, IDXW)],
                            acc64.at[idx_v.at[row]], add=True)
        return carry

    lax.fori_loop(0, NOUT, outer, 0)
    plsc.subcore_barrier()
    pltpu.sync_copy(acc64.at[pl.ds(pl.multiple_of(s * RPS, 8), RPS)],
                    s_out.at[c, pl.ds(pl.multiple_of(s * RPS, 8), RPS)])


def _sc_pool_body(af_hbm, cidx_hbm, g_out, cidx_v, gbuf, sem):
    c = lax.axis_index("c")
    s = lax.axis_index("s")
    w = c * SC_SUB + s
    pltpu.sync_copy(cidx_hbm.at[w], cidx_v)
    for i in range(PROWS):
        pltpu.async_copy(af_hbm.at[cidx_v.at[i]],
                         gbuf.at[pl.ds(i * PIDXW, PIDXW)], sem).wait()
    pltpu.sync_copy(gbuf, g_out.at[pl.ds(pl.multiple_of(w * PW, 8), PW)])


@functools.cache
def _get_sc_scatter_gather_cnt():
  return pl.kernel(
    _sc_scatter_gather_cnt_body,
    out_type=(
        jax.ShapeDtypeStruct((SC_CORES, NP, AFL), jnp.float32),
        jax.ShapeDtypeStruct((E, AFL), jnp.float32),
        jax.ShapeDtypeStruct((SC_CORES, NP, 16), jnp.float32),
    ),
    mesh=_mesh(),
    compiler_params=pltpu.CompilerParams(use_tc_tiling_on_sc=False),
    scratch_types=[
        pltpu.VMEM((NROW, IDXW), jnp.int32),
        pltpu.VMEM((CH, AFL), jnp.float32),
        pltpu.VMEM((CH, AFL), jnp.float32),
        pltpu.VMEM((IDXW, 16), jnp.float32),
        pltpu.SemaphoreType.DMA,
        pltpu.VMEM_SHARED((NP, AFL), jnp.float32),
        pltpu.VMEM_SHARED((NP, 16), jnp.float32),
    ],
)

@functools.cache
def _get_sc_scatter_gather():
  return pl.kernel(
    _sc_scatter_gather_body,
    out_type=(
        jax.ShapeDtypeStruct((SC_CORES, NP, AFL), jnp.float32),
        jax.ShapeDtypeStruct((E, AFL), jnp.float32),
    ),
    mesh=_mesh(),
    compiler_params=pltpu.CompilerParams(use_tc_tiling_on_sc=False),
    scratch_types=[
        pltpu.VMEM((NROW, IDXW), jnp.int32),
        pltpu.VMEM((CH, AFL), jnp.float32),
        pltpu.VMEM((CH, AFL), jnp.float32),
        pltpu.SemaphoreType.DMA,
        pltpu.VMEM_SHARED((NP, AFL), jnp.float32),
    ],
)

@functools.cache
def _get_sc_scatter():
  return pl.kernel(
    _sc_scatter_body,
    out_type=jax.ShapeDtypeStruct((SC_CORES, NP, AFL), jnp.float32),
    mesh=_mesh(),
    compiler_params=pltpu.CompilerParams(use_tc_tiling_on_sc=False),
    scratch_types=[
        pltpu.VMEM((NROW, IDXW), jnp.int32),
        pltpu.VMEM((CH, AFL), jnp.float32),
        pltpu.VMEM_SHARED((NP, AFL), jnp.float32),
    ],
)

@functools.cache
def _get_sc_pool():
  return pl.kernel(
    _sc_pool_body,
    out_type=jax.ShapeDtypeStruct((NC * NA, AFL), jnp.float32),
    mesh=_mesh(),
    compiler_params=pltpu.CompilerParams(use_tc_tiling_on_sc=False),
    scratch_types=[
        pltpu.VMEM((PROWS, PIDXW), jnp.int32),
        pltpu.VMEM((PW, AFL), jnp.float32),
        pltpu.SemaphoreType.DMA,
    ],
)


def _sc_scatter_gather_cnt(*args):
    return _get_sc_scatter_gather_cnt()(*args)


def _sc_scatter_gather(*args):
    return _get_sc_scatter_gather()(*args)


def _sc_scatter(*args):
    return _get_sc_scatter()(*args)


def _sc_pool(*args):
    return _get_sc_pool()(*args)


def kernel(atom_fea, nbr_fea, nbr_fea_idx, crystal_atom_idx, params):
    p = params
    tgt3d = nbr_fea_idx.astype(jnp.int32).reshape(NW, NROW, IDXW)
    cidx3d = crystal_atom_idx.astype(jnp.int32).reshape(NW, PROWS, PIDXW)
    z64 = jnp.zeros((NP, AFL), jnp.float32)
    z16 = jnp.zeros((NP, 16), jnp.float32)
    ones16 = jnp.ones((IDXW, 16), jnp.float32)
    nbr8 = nbr_fea.reshape(E // 8, 8 * NBR)

    af = _embed(atom_fea, p['W_emb'], p['b_emb'])
    cnt_p = None
    for l in range(3):
        t, ezs = _edge_mlp(nbr8,
                           p['c%d_W1' % l], p['c%d_b1' % l],
                           p['c%d_W2' % l], p['c%d_b2' % l],
                           p['c%d_W3' % l], p['c%d_b3' % l])
        x = _bn(af)
        if l == 0:
            ezt_p, xi, cnt_p = _sc_scatter_gather_cnt(
                t, x, tgt3d, z64, z16, ones16)
        else:
            ezt_p, xi = _sc_scatter_gather(t, x, tgt3d, z64)
        comb = _combine(x, ezs, ezt_p)
        msg = _msg(xi, comb, p['c%d_We1' % l], p['c%d_be1' % l],
                   p['c%d_We2' % l], p['c%d_be2' % l])
        s_p = _sc_scatter(msg, tgt3d, z64)
        af = _tail(s_p, cnt_p, comb)

    gathered = _sc_pool(af, cidx3d)
    return _head(gathered, p)


# speedup vs baseline: 2.8811x; 1.1608x over previous
import functools

import jax
import jax.numpy as jnp
from jax import lax
from jax.experimental import pallas as pl
from jax.experimental.pallas import tpu as pltpu
from jax.experimental.pallas import tpu_sc as plsc

N = 10000
M = 32
E = N * M
ORIG = 128
NBR = 16
AFL = 64
NC = 128
NA = 78

SC_CORES = 2
SC_SUB = 16
NW = SC_CORES * SC_SUB
EW = E // NW
IDXW = 50
NROW = EW // IDXW
GRP = 4
NOUT = NROW // GRP
CH = IDXW * GRP
NP = 10240
RPS = NP // SC_SUB

PIDXW = 104
PROWS = 3
PW = PIDXW * PROWS


def _mm(a, b):
    return _dot(a.astype(jnp.bfloat16), b.astype(jnp.bfloat16))


def _dot(a, b):
    return lax.dot_general(a, b, (((1,), (0,)), ((), ())),
                           preferred_element_type=jnp.float32)


def _mm3(a, bh, bl):
    del bl
    return _dot(a.astype(jnp.bfloat16), bh)


def _split(w):
    hi = w.astype(jnp.bfloat16)
    lo = (w - hi.astype(jnp.float32)).astype(jnp.bfloat16)
    return hi, lo


def _blkdiag(w, k):
    kk, jj = w.shape
    eye = jnp.eye(k, dtype=w.dtype)
    return (eye[:, None, :, None] * w[None, :, None, :]).reshape(k * kk, k * jj)


def _embed_body(a_ref, w_ref, b_ref, o_ref):
    o_ref[...] = _mm(a_ref[...], w_ref[...]) + b_ref[...]


def _embed(atom_fea, w, b):
    return pl.pallas_call(
        _embed_body,
        grid=(10,),
        in_specs=[
            pl.BlockSpec((N // 10, ORIG), lambda i: (i, 0)),
            pl.BlockSpec((ORIG, AFL), lambda i: (0, 0)),
            pl.BlockSpec((1, AFL), lambda i: (0, 0)),
        ],
        out_specs=pl.BlockSpec((N // 10, AFL), lambda i: (i, 0)),
        out_shape=jax.ShapeDtypeStruct((N, AFL), jnp.float32),
    )(atom_fea, w, b.reshape(1, AFL))


_BE = 1280
_GA = _BE // M


def _edge_mlp_body(nbr_ref, w1, b1, w2, b2, w3, b3, t_ref, ezs_ref):
    h = jnp.maximum(_mm(nbr_ref[...], w1[...]) + b1[...], 0.0)
    h = jnp.maximum(_mm(h, w2[...]) + b2[...], 0.0)
    t = _mm(h, w3[...]) + b3[...]
    t_ref[...] = t
    ezs_ref[...] = jnp.sum(t.reshape(_GA, M, AFL), axis=1)


def _edge_mlp(nbr_flat, w1, b1, w2, b2, w3, b3):
    return pl.pallas_call(
        _edge_mlp_body,
        grid=(E // _BE,),
        in_specs=[
            pl.BlockSpec((_BE, NBR), lambda i: (i, 0)),
            pl.BlockSpec((NBR, 256), lambda i: (0, 0)),
            pl.BlockSpec((1, 256), lambda i: (0, 0)),
            pl.BlockSpec((256, 128), lambda i: (0, 0)),
            pl.BlockSpec((1, 128), lambda i: (0, 0)),
            pl.BlockSpec((128, AFL), lambda i: (0, 0)),
            pl.BlockSpec((1, AFL), lambda i: (0, 0)),
        ],
        out_specs=[
            pl.BlockSpec((_BE, AFL), lambda i: (i, 0)),
            pl.BlockSpec((_GA, AFL), lambda i: (i, 0)),
        ],
        out_shape=[
            jax.ShapeDtypeStruct((E, AFL), jnp.float32),
            jax.ShapeDtypeStruct((N, AFL), jnp.float32),
        ],
    )(nbr_flat, w1, b1.reshape(1, 256), w2, b2.reshape(1, 128), w3,
      b3.reshape(1, AFL))


def _bn_body(a_ref, x_ref):
    a = a_ref[...]
    m = jnp.mean(a, axis=0, keepdims=True)
    v = jnp.mean((a - m) * (a - m), axis=0, keepdims=True)
    x_ref[...] = (a - m) * lax.rsqrt(v + 1e-5)


def _bn(a):
    return pl.pallas_call(
        _bn_body,
        out_shape=jax.ShapeDtypeStruct((N, AFL), jnp.float32),
    )(a)


def _comb_body(x_ref, ezs_ref, ezt_ref, o_ref):
    ezt = ezt_ref[...]
    o_ref[...] = x_ref[...] + ezs_ref[...] + ezt[0, :N] + ezt[1, :N]


def _combine(x, ezs, ezt_p):
    return pl.pallas_call(
        _comb_body,
        out_shape=jax.ShapeDtypeStruct((N, AFL), jnp.float32),
    )(x, ezs, ezt_p)


def _msg_body(xi_ref, comb_ref, we1, be1, we2, be2, o_ref):
    xi = xi_ref[...]
    xj = jnp.broadcast_to(comb_ref[...].reshape(_GA, 1, AFL),
                          (_GA, M, AFL)).reshape(_BE, AFL)
    cat = jnp.concatenate([xi, xj - xi], axis=1)
    h = jnp.maximum(_mm(cat, we1[...]) + be1[...], 0.0)
    o_ref[...] = _mm(h, we2[...]) + be2[...]


def _msg(xi, comb, we1, be1, we2, be2):
    return pl.pallas_call(
        _msg_body,
        grid=(E // _BE,),
        in_specs=[
            pl.BlockSpec((_BE, AFL), lambda i: (i, 0)),
            pl.BlockSpec((_GA, AFL), lambda i: (i, 0)),
            pl.BlockSpec((2 * AFL, 256), lambda i: (0, 0)),
            pl.BlockSpec((1, 256), lambda i: (0, 0)),
            pl.BlockSpec((256, AFL), lambda i: (0, 0)),
            pl.BlockSpec((1, AFL), lambda i: (0, 0)),
        ],
        out_specs=pl.BlockSpec((_BE, AFL), lambda i: (i, 0)),
        out_shape=jax.ShapeDtypeStruct((E, AFL), jnp.float32),
    )(xi, comb, we1, be1.reshape(1, 256), we2, be2.reshape(1, AFL))


def _tail_body(s_ref, cnt_ref, comb_ref, o_ref):
    cnt_p = cnt_ref[...]
    cnt = jnp.maximum(cnt_p[0, :N, :1] + cnt_p[1, :N, :1], 1.0)
    s_p = s_ref[...]
    agg = (s_p[0, :N] + s_p[1, :N]) / cnt
    m = jnp.mean(agg, axis=0, keepdims=True)
    v = jnp.mean((agg - m) * (agg - m), axis=0, keepdims=True)
    o_ref[...] = (agg - m) * lax.rsqrt(v + 1e-5) + comb_ref[...]


def _tail(s_p, cnt_p, comb):
    return pl.pallas_call(
        _tail_body,
        out_shape=jax.ShapeDtypeStruct((N, AFL), jnp.float32),
    )(s_p, cnt_p, comb)


def _head_body(g_ref, wfc, bfc, wh1, bh1, wh2, bh2, wo, bo, o_ref):
    pooled = jnp.mean(g_ref[...], axis=1)
    h = _mm(pooled, wfc[...]) + bfc[...]
    h = jnp.maximum(_mm(h, wh1[...]) + bh1[...], 0.0)
    h = jnp.maximum(_mm(h, wh2[...]) + bh2[...], 0.0)
    o_ref[...] = _mm(h, wo[...]) + bo[...]


def _head(gathered, p):
    return pl.pallas_call(
        _head_body,
        out_shape=jax.ShapeDtypeStruct((NC, 1), jnp.float32),
    )(gathered.reshape(NC, NA, AFL), p['W_fc'], p['b_fc'].reshape(1, 256),
      p['Wh1'], p['bh1'].reshape(1, 128), p['Wh2'], p['bh2'].reshape(1, 64),
      p['Wout'], p['bout'].reshape(1, 1))


@functools.cache
def _mesh():
    return plsc.VectorSubcoreMesh(core_axis_name="c", subcore_axis_name="s",
                                  num_cores=SC_CORES, num_subcores=SC_SUB)


def _zero_acc(z, acc, s):
    off = pl.multiple_of(s * RPS, 8)
    pltpu.sync_copy(z.at[pl.ds(off, RPS)], acc.at[pl.ds(off, RPS)])


def _dump_acc(acc, out, c, s):
    off = pl.multiple_of(s * RPS, 8)
    pltpu.sync_copy(acc.at[pl.ds(off, RPS)], out.at[c, pl.ds(off, RPS)])


def _sc_scatter_gather_cnt_body(t_hbm, x_hbm, idx_hbm, z64, z16, ones_hbm,
                                ezt_out, xi_out, cnt_out,
                                idx_v, tbuf0, tbuf1, xbuf0, xbuf1, ones_v,
                                semt0, semt1, semg0, semg1, sems,
                                acc64, acc16):
    c = lax.axis_index("c")
    s = lax.axis_index("s")
    w = c * SC_SUB + s
    pltpu.sync_copy(idx_hbm.at[w], idx_v)
    pltpu.sync_copy(ones_hbm, ones_v)
    _zero_acc(z64, acc64, s)
    _zero_acc(z16, acc16, s)
    plsc.subcore_barrier()

    tbufs = ((tbuf0, semt0), (tbuf1, semt1))
    xbufs = ((xbuf0, semg0), (xbuf1, semg1))
    base0 = pl.multiple_of(w * EW, 8)
    pltpu.async_copy(t_hbm.at[pl.ds(base0, CH)], tbuf0, semt0)
    pltpu.async_copy(t_hbm.at[pl.ds(base0 + CH, CH)], tbuf1, semt1)

    def do_chunk(j, b, prefetch):
        tb, st = tbufs[b]
        xb, sg = xbufs[b]
        pltpu.make_async_copy(t_hbm.at[pl.ds(0, CH)], tb, st).wait()
        gd = []
        sd = []
        for i in range(GRP):
            row = j * GRP + i
            gd.append(pltpu.async_copy(x_hbm.at[idx_v.at[row]],
                                       xb.at[pl.ds(i * IDXW, IDXW)], sg))
            sd.append(pltpu.async_copy(tb.at[pl.ds(i * IDXW, IDXW)],
                                       acc64.at[idx_v.at[row]], sems,
                                       add=True))
            sd.append(pltpu.async_copy(ones_v, acc16.at[idx_v.at[row]],
                                       sems, add=True))
        for d in sd:
            d.wait()
        if prefetch:
            nxt = j + 2

            @pl.when(nxt < NOUT)
            def _():
                nb = pl.multiple_of(w * EW + nxt * CH, 8)
                pltpu.async_copy(t_hbm.at[pl.ds(nb, CH)], tb, st)

        for d in gd:
            d.wait()
        base = pl.multiple_of(w * EW + j * CH, 8)
        pltpu.sync_copy(xb, xi_out.at[pl.ds(base, CH)])

    def outer(k, carry):
        do_chunk(2 * k, 0, True)
        do_chunk(2 * k + 1, 1, True)
        return carry

    lax.fori_loop(0, NOUT // 2, outer, 0)
    plsc.subcore_barrier()
    _dump_acc(acc64, ezt_out, c, s)
    _dump_acc(acc16, cnt_out, c, s)


def _sc_scatter_gather_body(t_hbm, x_hbm, idx_hbm, z64,
                            ezt_out, xi_out,
                            idx_v, tbuf0, tbuf1, xbuf0, xbuf1,
                            semt0, semt1, semg0, semg1, sems,
                            acc64):
    c = lax.axis_index("c")
    s = lax.axis_index("s")
    w = c * SC_SUB + s
    pltpu.sync_copy(idx_hbm.at[w], idx_v)
    _zero_acc(z64, acc64, s)
    plsc.subcore_barrier()

    tbufs = ((tbuf0, semt0), (tbuf1, semt1))
    xbufs = ((xbuf0, semg0), (xbuf1, semg1))
    base0 = pl.multiple_of(w * EW, 8)
    pltpu.async_copy(t_hbm.at[pl.ds(base0, CH)], tbuf0, semt0)
    pltpu.async_copy(t_hbm.at[pl.ds(base0 + CH, CH)], tbuf1, semt1)

    def do_chunk(j, b, prefetch):
        tb, st = tbufs[b]
        xb, sg = xbufs[b]
        pltpu.make_async_copy(t_hbm.at[pl.ds(0, CH)], tb, st).wait()
        gd = []
        sd = []
        for i in range(GRP):
            row = j * GRP + i
            gd.append(pltpu.async_copy(x_hbm.at[idx_v.at[row]],
                                       xb.at[pl.ds(i * IDXW, IDXW)], sg))
            sd.append(pltpu.async_copy(tb.at[pl.ds(i * IDXW, IDXW)],
                                       acc64.at[idx_v.at[row]], sems,
                                       add=True))
        for d in sd:
            d.wait()
        if prefetch:
            nxt = j + 2

            @pl.when(nxt < NOUT)
            def _():
                nb = pl.multiple_of(w * EW + nxt * CH, 8)
                pltpu.async_copy(t_hbm.at[pl.ds(nb, CH)], tb, st)

        for d in gd:
            d.wait()
        base = pl.multiple_of(w * EW + j * CH, 8)
        pltpu.sync_copy(xb, xi_out.at[pl.ds(base, CH)])

    def outer(k, carry):
        do_chunk(2 * k, 0, True)
        do_chunk(2 * k + 1, 1, True)
        return carry

    lax.fori_loop(0, NOUT // 2, outer, 0)
    plsc.subcore_barrier()
    _dump_acc(acc64, ezt_out, c, s)


def _sc_scatter_body(v_hbm, idx_hbm, z64, s_out,
                     idx_v, vbuf0, vbuf1, semt0, semt1, sems, acc64):
    c = lax.axis_index("c")
    s = lax.axis_index("s")
    w = c * SC_SUB + s
    pltpu.sync_copy(idx_hbm.at[w], idx_v)
    _zero_acc(z64, acc64, s)
    plsc.subcore_barrier()

    vbufs = ((vbuf0, semt0), (vbuf1, semt1))
    base0 = pl.multiple_of(w * EW, 8)
    pltpu.async_copy(v_hbm.at[pl.ds(base0, CH)], vbuf0, semt0)
    pltpu.async_copy(v_hbm.at[pl.ds(base0 + CH, CH)], vbuf1, semt1)

    def do_chunk(j, b, prefetch):
        vb, st = vbufs[b]
        pltpu.make_async_copy(v_hbm.at[pl.ds(0, CH)], vb, st).wait()
        sd = []
        for i in range(GRP):
            row = j * GRP + i
            sd.append(pltpu.async_copy(vb.at[pl.ds(i * IDXW, IDXW)],
                                       acc64.at[idx_v.at[row]], sems,
                                       add=True))
        for d in sd:
            d.wait()
        if prefetch:
            nxt = j + 2

            @pl.when(nxt < NOUT)
            def _():
                nb = pl.multiple_of(w * EW + nxt * CH, 8)
                pltpu.async_copy(v_hbm.at[pl.ds(nb, CH)], vb, st)

    def outer(k, carry):
        do_chunk(2 * k, 0, True)
        do_chunk(2 * k + 1, 1, True)
        return carry

    lax.fori_loop(0, NOUT // 2, outer, 0)
    plsc.subcore_barrier()
    _dump_acc(acc64, s_out, c, s)


def _sc_pool_body(af_hbm, cidx_hbm, g_out, cidx_v, gbuf, sem):
    c = lax.axis_index("c")
    s = lax.axis_index("s")
    w = c * SC_SUB + s
    pltpu.sync_copy(cidx_hbm.at[w], cidx_v)
    for i in range(PROWS):
        pltpu.async_copy(af_hbm.at[cidx_v.at[i]],
                         gbuf.at[pl.ds(i * PIDXW, PIDXW)], sem).wait()
    pltpu.sync_copy(gbuf, g_out.at[pl.ds(pl.multiple_of(w * PW, 8), PW)])


@functools.cache
def _get_sc_scatter_gather_cnt():
  return pl.kernel(
    _sc_scatter_gather_cnt_body,
    out_type=(
        jax.ShapeDtypeStruct((SC_CORES, NP, AFL), jnp.float32),
        jax.ShapeDtypeStruct((E, AFL), jnp.float32),
        jax.ShapeDtypeStruct((SC_CORES, NP, 16), jnp.float32),
    ),
    mesh=_mesh(),
    compiler_params=pltpu.CompilerParams(use_tc_tiling_on_sc=False),
    scratch_types=[
        pltpu.VMEM((NROW, IDXW), jnp.int32),
        pltpu.VMEM((CH, AFL), jnp.float32),
        pltpu.VMEM((CH, AFL), jnp.float32),
        pltpu.VMEM((CH, AFL), jnp.float32),
        pltpu.VMEM((CH, AFL), jnp.float32),
        pltpu.VMEM((IDXW, 16), jnp.float32),
        pltpu.SemaphoreType.DMA,
        pltpu.SemaphoreType.DMA,
        pltpu.SemaphoreType.DMA,
        pltpu.SemaphoreType.DMA,
        pltpu.SemaphoreType.DMA,
        pltpu.VMEM_SHARED((NP, AFL), jnp.float32),
        pltpu.VMEM_SHARED((NP, 16), jnp.float32),
    ],
)

@functools.cache
def _get_sc_scatter_gather():
  return pl.kernel(
    _sc_scatter_gather_body,
    out_type=(
        jax.ShapeDtypeStruct((SC_CORES, NP, AFL), jnp.float32),
        jax.ShapeDtypeStruct((E, AFL), jnp.float32),
    ),
    mesh=_mesh(),
    compiler_params=pltpu.CompilerParams(use_tc_tiling_on_sc=False),
    scratch_types=[
        pltpu.VMEM((NROW, IDXW), jnp.int32),
        pltpu.VMEM((CH, AFL), jnp.float32),
        pltpu.VMEM((CH, AFL), jnp.float32),
        pltpu.VMEM((CH, AFL), jnp.float32),
        pltpu.VMEM((CH, AFL), jnp.float32),
        pltpu.SemaphoreType.DMA,
        pltpu.SemaphoreType.DMA,
        pltpu.SemaphoreType.DMA,
        pltpu.SemaphoreType.DMA,
        pltpu.SemaphoreType.DMA,
        pltpu.VMEM_SHARED((NP, AFL), jnp.float32),
    ],
)

@functools.cache
def _get_sc_scatter():
  return pl.kernel(
    _sc_scatter_body,
    out_type=jax.ShapeDtypeStruct((SC_CORES, NP, AFL), jnp.float32),
    mesh=_mesh(),
    compiler_params=pltpu.CompilerParams(use_tc_tiling_on_sc=False),
    scratch_types=[
        pltpu.VMEM((NROW, IDXW), jnp.int32),
        pltpu.VMEM((CH, AFL), jnp.float32),
        pltpu.VMEM((CH, AFL), jnp.float32),
        pltpu.SemaphoreType.DMA,
        pltpu.SemaphoreType.DMA,
        pltpu.SemaphoreType.DMA,
        pltpu.VMEM_SHARED((NP, AFL), jnp.float32),
    ],
)

@functools.cache
def _get_sc_pool():
  return pl.kernel(
    _sc_pool_body,
    out_type=jax.ShapeDtypeStruct((NC * NA, AFL), jnp.float32),
    mesh=_mesh(),
    compiler_params=pltpu.CompilerParams(use_tc_tiling_on_sc=False),
    scratch_types=[
        pltpu.VMEM((PROWS, PIDXW), jnp.int32),
        pltpu.VMEM((PW, AFL), jnp.float32),
        pltpu.SemaphoreType.DMA,
    ],
)


def _sc_scatter_gather_cnt(*args):
    return _get_sc_scatter_gather_cnt()(*args)


def _sc_scatter_gather(*args):
    return _get_sc_scatter_gather()(*args)


def _sc_scatter(*args):
    return _get_sc_scatter()(*args)


def _sc_pool(*args):
    return _get_sc_pool()(*args)


def kernel(atom_fea, nbr_fea, nbr_fea_idx, crystal_atom_idx, params):
    p = params
    tgt3d = nbr_fea_idx.astype(jnp.int32).reshape(NW, NROW, IDXW)
    cidx3d = crystal_atom_idx.astype(jnp.int32).reshape(NW, PROWS, PIDXW)
    z64 = jnp.zeros((NP, AFL), jnp.float32)
    z16 = jnp.zeros((NP, 16), jnp.float32)
    ones16 = jnp.ones((IDXW, 16), jnp.float32)
    nbr_flat = nbr_fea.reshape(E, NBR)

    af = _embed(atom_fea, p['W_emb'], p['b_emb'])
    cnt_p = None
    for l in range(3):
        t, ezs = _edge_mlp(nbr_flat,
                           p['c%d_W1' % l], p['c%d_b1' % l],
                           p['c%d_W2' % l], p['c%d_b2' % l],
                           p['c%d_W3' % l], p['c%d_b3' % l])
        x = _bn(af)
        if l == 0:
            ezt_p, xi, cnt_p = _sc_scatter_gather_cnt(
                t, x, tgt3d, z64, z16, ones16)
        else:
            ezt_p, xi = _sc_scatter_gather(t, x, tgt3d, z64)
        comb = _combine(x, ezs, ezt_p)
        msg = _msg(xi, comb, p['c%d_We1' % l], p['c%d_be1' % l],
                   p['c%d_We2' % l], p['c%d_be2' % l])
        s_p = _sc_scatter(msg, tgt3d, z64)
        af = _tail(s_p, cnt_p, comb)

    gathered = _sc_pool(af, cidx3d)
    return _head(gathered, p)


# speedup vs baseline: 3.0419x; 1.0558x over previous
import functools

import jax
import jax.numpy as jnp
from jax import lax
from jax.experimental import pallas as pl
from jax.experimental.pallas import tpu as pltpu
from jax.experimental.pallas import tpu_sc as plsc

N = 10000
M = 32
E = N * M
ORIG = 128
NBR = 16
AFL = 64
NC = 128
NA = 78

SC_CORES = 2
SC_SUB = 16
NW = SC_CORES * SC_SUB
EW = E // NW
IDXW = 50
NROW = EW // IDXW
GRP = 4
NOUT = NROW // GRP
CH = IDXW * GRP
NP = 10240
RPS = NP // SC_SUB

PIDXW = 104
PROWS = 3
PW = PIDXW * PROWS


def _mm(a, b):
    return _dot(a.astype(jnp.bfloat16), b.astype(jnp.bfloat16))


def _dot(a, b):
    return lax.dot_general(a, b, (((1,), (0,)), ((), ())),
                           preferred_element_type=jnp.float32)


def _mm3(a, bh, bl):
    del bl
    return _dot(a.astype(jnp.bfloat16), bh)


def _split(w):
    hi = w.astype(jnp.bfloat16)
    lo = (w - hi.astype(jnp.float32)).astype(jnp.bfloat16)
    return hi, lo


def _blkdiag(w, k):
    kk, jj = w.shape
    eye = jnp.eye(k, dtype=w.dtype)
    return (eye[:, None, :, None] * w[None, :, None, :]).reshape(k * kk, k * jj)


def _embed_body(a_ref, w_ref, b_ref, o_ref):
    af = _mm(a_ref[...], w_ref[...]) + b_ref[...]
    o_ref[...] = jnp.concatenate([af, jnp.zeros_like(af)], axis=1)


def _embed(atom_fea, w, b):
    return pl.pallas_call(
        _embed_body,
        grid=(10,),
        in_specs=[
            pl.BlockSpec((N // 10, ORIG), lambda i: (i, 0)),
            pl.BlockSpec((ORIG, AFL), lambda i: (0, 0)),
            pl.BlockSpec((1, AFL), lambda i: (0, 0)),
        ],
        out_specs=pl.BlockSpec((N // 10, 2 * AFL), lambda i: (i, 0)),
        out_shape=jax.ShapeDtypeStruct((N, 2 * AFL), jnp.float32),
    )(atom_fea, w, b.reshape(1, AFL))


_BE = 1280
_GA = _BE // M


def _edge_mlp_body(nbr_ref, w1, b1, w2, b2, w3, b3, t_ref, ezs_ref):
    h = jnp.maximum(_mm(nbr_ref[...], w1[...]) + b1[...], 0.0)
    h = jnp.maximum(_mm(h, w2[...]) + b2[...], 0.0)
    t = _mm(h, w3[...]) + b3[...]
    t_ref[...] = t
    ezs_ref[...] = jnp.sum(t.reshape(_GA, M, AFL), axis=1)


def _edge_mlp(nbr_flat, w1, b1, w2, b2, w3, b3):
    return pl.pallas_call(
        _edge_mlp_body,
        grid=(E // _BE,),
        in_specs=[
            pl.BlockSpec((_BE, NBR), lambda i: (i, 0)),
            pl.BlockSpec((NBR, 256), lambda i: (0, 0)),
            pl.BlockSpec((1, 256), lambda i: (0, 0)),
            pl.BlockSpec((256, 128), lambda i: (0, 0)),
            pl.BlockSpec((1, 128), lambda i: (0, 0)),
            pl.BlockSpec((128, AFL), lambda i: (0, 0)),
            pl.BlockSpec((1, AFL), lambda i: (0, 0)),
        ],
        out_specs=[
            pl.BlockSpec((_BE, AFL), lambda i: (i, 0)),
            pl.BlockSpec((_GA, AFL), lambda i: (i, 0)),
        ],
        out_shape=[
            jax.ShapeDtypeStruct((E, AFL), jnp.float32),
            jax.ShapeDtypeStruct((N, AFL), jnp.float32),
        ],
    )(nbr_flat, w1, b1.reshape(1, 256), w2, b2.reshape(1, 128), w3,
      b3.reshape(1, AFL))


def _bn_body(a_ref, x_ref):
    a = a_ref[...][:, :AFL]
    m = jnp.mean(a, axis=0, keepdims=True)
    v = jnp.mean((a - m) * (a - m), axis=0, keepdims=True)
    xn = (a - m) * lax.rsqrt(v + 1e-5)
    x_ref[...] = jnp.concatenate([xn, jnp.zeros_like(xn)], axis=1)


def _bn(a):
    return pl.pallas_call(
        _bn_body,
        out_shape=jax.ShapeDtypeStruct((N, 2 * AFL), jnp.float32),
    )(a)


def _comb_body(x_ref, ezs_ref, ezt_ref, o_ref):
    ezt = ezt_ref[...]
    o_ref[...] = (x_ref[...][:, :AFL] + ezs_ref[...]
                  + ezt[0, :N] + ezt[1, :N])


def _combine(x, ezs, ezt_p):
    return pl.pallas_call(
        _comb_body,
        out_shape=jax.ShapeDtypeStruct((N, AFL), jnp.float32),
    )(x, ezs, ezt_p)


def _msg_body(xi_ref, comb_ref, we1, be1, we2, be2, o_ref):
    xi = xi_ref[...][:, :AFL]
    xj = jnp.broadcast_to(comb_ref[...].reshape(_GA, 1, AFL),
                          (_GA, M, AFL)).reshape(_BE, AFL)
    cat = jnp.concatenate([xi, xj - xi], axis=1)
    h = jnp.maximum(_mm(cat, we1[...]) + be1[...], 0.0)
    o_ref[...] = _mm(h, we2[...]) + be2[...]


def _msg(xi, comb, we1, be1, we2, be2):
    return pl.pallas_call(
        _msg_body,
        grid=(E // _BE,),
        in_specs=[
            pl.BlockSpec((_BE, 2 * AFL), lambda i: (i, 0)),
            pl.BlockSpec((_GA, AFL), lambda i: (i, 0)),
            pl.BlockSpec((2 * AFL, 256), lambda i: (0, 0)),
            pl.BlockSpec((1, 256), lambda i: (0, 0)),
            pl.BlockSpec((256, AFL), lambda i: (0, 0)),
            pl.BlockSpec((1, AFL), lambda i: (0, 0)),
        ],
        out_specs=pl.BlockSpec((_BE, AFL), lambda i: (i, 0)),
        out_shape=jax.ShapeDtypeStruct((E, AFL), jnp.float32),
    )(xi, comb, we1, be1.reshape(1, 256), we2, be2.reshape(1, AFL))


def _tail_body(s_ref, cnt_ref, comb_ref, o_ref):
    cnt_p = cnt_ref[...]
    cnt = jnp.maximum(cnt_p[0, :N, :1] + cnt_p[1, :N, :1], 1.0)
    s_p = s_ref[...]
    agg = (s_p[0, :N] + s_p[1, :N]) / cnt
    m = jnp.mean(agg, axis=0, keepdims=True)
    v = jnp.mean((agg - m) * (agg - m), axis=0, keepdims=True)
    af = (agg - m) * lax.rsqrt(v + 1e-5) + comb_ref[...]
    o_ref[...] = jnp.concatenate([af, jnp.zeros_like(af)], axis=1)


def _tail(s_p, cnt_p, comb):
    return pl.pallas_call(
        _tail_body,
        out_shape=jax.ShapeDtypeStruct((N, 2 * AFL), jnp.float32),
    )(s_p, cnt_p, comb)


def _head_body(g_ref, wfc, bfc, wh1, bh1, wh2, bh2, wo, bo, o_ref):
    pooled = jnp.mean(g_ref[...][:, :, :AFL], axis=1)
    h = _mm(pooled, wfc[...]) + bfc[...]
    h = jnp.maximum(_mm(h, wh1[...]) + bh1[...], 0.0)
    h = jnp.maximum(_mm(h, wh2[...]) + bh2[...], 0.0)
    o_ref[...] = _mm(h, wo[...]) + bo[...]


def _head(gathered, p):
    return pl.pallas_call(
        _head_body,
        out_shape=jax.ShapeDtypeStruct((NC, 1), jnp.float32),
    )(gathered.reshape(NC, NA, 2 * AFL), p['W_fc'], p['b_fc'].reshape(1, 256),
      p['Wh1'], p['bh1'].reshape(1, 128), p['Wh2'], p['bh2'].reshape(1, 64),
      p['Wout'], p['bout'].reshape(1, 1))


@functools.cache
def _mesh():
    return plsc.VectorSubcoreMesh(core_axis_name="c", subcore_axis_name="s",
                                  num_cores=SC_CORES, num_subcores=SC_SUB)


def _zero_acc(z, acc, s):
    off = pl.multiple_of(s * RPS, 8)
    pltpu.sync_copy(z.at[pl.ds(off, RPS)], acc.at[pl.ds(off, RPS)])


def _dump_acc(acc, out, c, s):
    off = pl.multiple_of(s * RPS, 8)
    pltpu.sync_copy(acc.at[pl.ds(off, RPS)], out.at[c, pl.ds(off, RPS)])


def _sc_cnt_body(idx_hbm, z16, ones_hbm, cnt_out,
                 idx_v, ones_v, sems, acc16):
    c = lax.axis_index("c")
    s = lax.axis_index("s")
    w = c * SC_SUB + s
    pltpu.sync_copy(idx_hbm.at[w], idx_v)
    pltpu.sync_copy(ones_hbm, ones_v)
    _zero_acc(z16, acc16, s)
    plsc.subcore_barrier()

    def chunk(j, carry):
        sd = []
        for i in range(GRP):
            row = j * GRP + i
            sd.append(pltpu.async_copy(ones_v, acc16.at[idx_v.at[row]],
                                       sems, add=True))
        for d in sd:
            d.wait()
        return carry

    lax.fori_loop(0, NOUT, chunk, 0)
    plsc.subcore_barrier()
    _dump_acc(acc16, cnt_out, c, s)


def _sc_scatter_gather_body(t_hbm, x_hbm, idx_hbm, z64,
                            ezt_out, xi_out,
                            idx_v, tbuf, xbuf, semt, semg, sems,
                            acc64):
    c = lax.axis_index("c")
    s = lax.axis_index("s")
    w = c * SC_SUB + s
    pltpu.sync_copy(idx_hbm.at[w], idx_v)
    _zero_acc(z64, acc64, s)
    plsc.subcore_barrier()

    base0 = pl.multiple_of(w * EW, 8)
    pltpu.async_copy(t_hbm.at[pl.ds(base0, CH)], tbuf, semt)

    def chunk(j, carry):
        pltpu.make_async_copy(t_hbm.at[pl.ds(0, CH)], tbuf, semt).wait()
        gd = []
        sd = []
        for i in range(GRP):
            row = j * GRP + i
            gd.append(pltpu.async_copy(x_hbm.at[idx_v.at[row]],
                                      xbuf.at[pl.ds(i * IDXW, IDXW)], semg))
            sd.append(pltpu.async_copy(tbuf.at[pl.ds(i * IDXW, IDXW)],
                                       acc64.at[idx_v.at[row]], sems,
                                       add=True))
        for d in sd:
            d.wait()
        nxt = j + 1

        @pl.when(nxt < NOUT)
        def _():
            nb = pl.multiple_of(w * EW + nxt * CH, 8)
            pltpu.async_copy(t_hbm.at[pl.ds(nb, CH)], tbuf, semt)

        for d in gd:
            d.wait()
        base = pl.multiple_of(w * EW + j * CH, 8)
        pltpu.sync_copy(xbuf, xi_out.at[pl.ds(base, CH)])
        return carry

    lax.fori_loop(0, NOUT, chunk, 0)
    plsc.subcore_barrier()
    _dump_acc(acc64, ezt_out, c, s)


def _sc_scatter_body(v_hbm, idx_hbm, z64, s_out,
                     idx_v, vbuf0, vbuf1, semt0, semt1, sems, acc64):
    c = lax.axis_index("c")
    s = lax.axis_index("s")
    w = c * SC_SUB + s
    pltpu.sync_copy(idx_hbm.at[w], idx_v)
    _zero_acc(z64, acc64, s)
    plsc.subcore_barrier()

    vbufs = ((vbuf0, semt0), (vbuf1, semt1))
    base0 = pl.multiple_of(w * EW, 8)
    pltpu.async_copy(v_hbm.at[pl.ds(base0, CH)], vbuf0, semt0)
    pltpu.async_copy(v_hbm.at[pl.ds(base0 + CH, CH)], vbuf1, semt1)

    def do_chunk(j, b, prefetch):
        vb, st = vbufs[b]
        pltpu.make_async_copy(v_hbm.at[pl.ds(0, CH)], vb, st).wait()
        sd = []
        for i in range(GRP):
            row = j * GRP + i
            sd.append(pltpu.async_copy(vb.at[pl.ds(i * IDXW, IDXW)],
                                       acc64.at[idx_v.at[row]], sems,
                                       add=True))
        for d in sd:
            d.wait()
        if prefetch:
            nxt = j + 2

            @pl.when(nxt < NOUT)
            def _():
                nb = pl.multiple_of(w * EW + nxt * CH, 8)
                pltpu.async_copy(v_hbm.at[pl.ds(nb, CH)], vb, st)

    def outer(k, carry):
        do_chunk(2 * k, 0, True)
        do_chunk(2 * k + 1, 1, True)
        return carry

    lax.fori_loop(0, NOUT // 2, outer, 0)
    plsc.subcore_barrier()
    _dump_acc(acc64, s_out, c, s)


def _sc_pool_body(af_hbm, cidx_hbm, g_out, cidx_v, gbuf, sem):
    c = lax.axis_index("c")
    s = lax.axis_index("s")
    w = c * SC_SUB + s
    pltpu.sync_copy(cidx_hbm.at[w], cidx_v)
    for i in range(PROWS):
        pltpu.async_copy(af_hbm.at[cidx_v.at[i]],
                         gbuf.at[pl.ds(i * PIDXW, PIDXW)], sem).wait()
    pltpu.sync_copy(gbuf, g_out.at[pl.ds(pl.multiple_of(w * PW, 8), PW)])


@functools.cache
def _get_sc_cnt():
  return pl.kernel(
    _sc_cnt_body,
    out_type=jax.ShapeDtypeStruct((SC_CORES, NP, 16), jnp.float32),
    mesh=_mesh(),
    compiler_params=pltpu.CompilerParams(use_tc_tiling_on_sc=False),
    scratch_types=[
        pltpu.VMEM((NROW, IDXW), jnp.int32),
        pltpu.VMEM((IDXW, 16), jnp.float32),
        pltpu.SemaphoreType.DMA,
        pltpu.VMEM_SHARED((NP, 16), jnp.float32),
    ],
)

@functools.cache
def _get_sc_scatter_gather():
  return pl.kernel(
    _sc_scatter_gather_body,
    out_type=(
        jax.ShapeDtypeStruct((SC_CORES, NP, AFL), jnp.float32),
        jax.ShapeDtypeStruct((E, 2 * AFL), jnp.float32),
    ),
    mesh=_mesh(),
    compiler_params=pltpu.CompilerParams(use_tc_tiling_on_sc=False),
    scratch_types=[
        pltpu.VMEM((NROW, IDXW), jnp.int32),
        pltpu.VMEM((CH, AFL), jnp.float32),
        pltpu.VMEM((CH, 2 * AFL), jnp.float32),
        pltpu.SemaphoreType.DMA,
        pltpu.SemaphoreType.DMA,
        pltpu.SemaphoreType.DMA,
        pltpu.VMEM_SHARED((NP, AFL), jnp.float32),
    ],
)

@functools.cache
def _get_sc_scatter():
  return pl.kernel(
    _sc_scatter_body,
    out_type=jax.ShapeDtypeStruct((SC_CORES, NP, AFL), jnp.float32),
    mesh=_mesh(),
    compiler_params=pltpu.CompilerParams(use_tc_tiling_on_sc=False),
    scratch_types=[
        pltpu.VMEM((NROW, IDXW), jnp.int32),
        pltpu.VMEM((CH, AFL), jnp.float32),
        pltpu.VMEM((CH, AFL), jnp.float32),
        pltpu.SemaphoreType.DMA,
        pltpu.SemaphoreType.DMA,
        pltpu.SemaphoreType.DMA,
        pltpu.VMEM_SHARED((NP, AFL), jnp.float32),
    ],
)

@functools.cache
def _get_sc_pool():
  return pl.kernel(
    _sc_pool_body,
    out_type=jax.ShapeDtypeStruct((NC * NA, 2 * AFL), jnp.float32),
    mesh=_mesh(),
    compiler_params=pltpu.CompilerParams(use_tc_tiling_on_sc=False),
    scratch_types=[
        pltpu.VMEM((PROWS, PIDXW), jnp.int32),
        pltpu.VMEM((PW, 2 * AFL), jnp.float32),
        pltpu.SemaphoreType.DMA,
    ],
)


def _sc_cnt(*args):
    return _get_sc_cnt()(*args)


def _sc_scatter_gather(*args):
    return _get_sc_scatter_gather()(*args)


def _sc_scatter(*args):
    return _get_sc_scatter()(*args)


def _sc_pool(*args):
    return _get_sc_pool()(*args)


def kernel(atom_fea, nbr_fea, nbr_fea_idx, crystal_atom_idx, params):
    p = params
    tgt3d = nbr_fea_idx.astype(jnp.int32).reshape(NW, NROW, IDXW)
    cidx3d = crystal_atom_idx.astype(jnp.int32).reshape(NW, PROWS, PIDXW)
    z64 = jnp.zeros((NP, AFL), jnp.float32)
    z16 = jnp.zeros((NP, 16), jnp.float32)
    ones16 = jnp.ones((IDXW, 16), jnp.float32)
    nbr_flat = nbr_fea.reshape(E, NBR)

    af = _embed(atom_fea, p['W_emb'], p['b_emb'])
    cnt_p = _sc_cnt(tgt3d, z16, ones16)
    for l in range(3):
        t, ezs = _edge_mlp(nbr_flat,
                           p['c%d_W1' % l], p['c%d_b1' % l],
                           p['c%d_W2' % l], p['c%d_b2' % l],
                           p['c%d_W3' % l], p['c%d_b3' % l])
        x = _bn(af)
        ezt_p, xi = _sc_scatter_gather(t, x, tgt3d, z64)
        comb = _combine(x, ezs, ezt_p)
        msg = _msg(xi, comb, p['c%d_We1' % l], p['c%d_be1' % l],
                   p['c%d_We2' % l], p['c%d_be2' % l])
        s_p = _sc_scatter(msg, tgt3d, z64)
        af = _tail(s_p, cnt_p, comb)

    gathered = _sc_pool(af, cidx3d)
    return _head(gathered, p)


# speedup vs baseline: 3.6642x; 1.2046x over previous
import functools

import jax
import jax.numpy as jnp
from jax import lax
from jax.experimental import pallas as pl
from jax.experimental.pallas import tpu as pltpu
from jax.experimental.pallas import tpu_sc as plsc

N = 10000
M = 32
E = N * M
ORIG = 128
NBR = 16
AFL = 64
NC = 128
NA = 78

SC_CORES = 2
SC_SUB = 16
NW = SC_CORES * SC_SUB
EW = E // NW
IDXW = 40
NROW = EW // IDXW
GRP = 2
NOUT = NROW // GRP
CH = IDXW * GRP
NP = 10240
RPS = NP // SC_SUB

PIDXW = 104
PROWS = 3
PW = PIDXW * PROWS


def _mm(a, b):
    return _dot(a.astype(jnp.bfloat16), b.astype(jnp.bfloat16))


def _dot(a, b):
    return lax.dot_general(a, b, (((1,), (0,)), ((), ())),
                           preferred_element_type=jnp.float32)


def _mm3(a, bh, bl):
    del bl
    return _dot(a.astype(jnp.bfloat16), bh)


def _split(w):
    hi = w.astype(jnp.bfloat16)
    lo = (w - hi.astype(jnp.float32)).astype(jnp.bfloat16)
    return hi, lo


def _blkdiag(w, k):
    kk, jj = w.shape
    eye = jnp.eye(k, dtype=w.dtype)
    return (eye[:, None, :, None] * w[None, :, None, :]).reshape(k * kk, k * jj)


def _embed_body(a_ref, w_ref, b_ref, o_ref):
    af = _mm(a_ref[...], w_ref[...]) + b_ref[...]
    o_ref[...] = jnp.concatenate([af, jnp.zeros_like(af)], axis=1)


def _embed(atom_fea, w, b):
    return pl.pallas_call(
        _embed_body,
        grid=(10,),
        in_specs=[
            pl.BlockSpec((N // 10, ORIG), lambda i: (i, 0)),
            pl.BlockSpec((ORIG, AFL), lambda i: (0, 0)),
            pl.BlockSpec((1, AFL), lambda i: (0, 0)),
        ],
        out_specs=pl.BlockSpec((N // 10, 2 * AFL), lambda i: (i, 0)),
        out_shape=jax.ShapeDtypeStruct((N, 2 * AFL), jnp.float32),
    )(atom_fea, w, b.reshape(1, AFL))


_BE = 1280
_GA = _BE // M


def _edge_mlp_body(nbr_ref, w1, b1, w2, b2, w3, b3, t_ref, ezs_ref):
    h = jnp.maximum(_mm(nbr_ref[...], w1[...]) + b1[...], 0.0)
    h = jnp.maximum(_mm(h, w2[...]) + b2[...], 0.0)
    t = _mm(h, w3[...]) + b3[...]
    t_ref[...] = jnp.concatenate([t, jnp.zeros_like(t)], axis=1)
    ezs_ref[...] = jnp.sum(t.reshape(_GA, M, AFL), axis=1)


def _edge_mlp(nbr_flat, w1, b1, w2, b2, w3, b3):
    return pl.pallas_call(
        _edge_mlp_body,
        grid=(E // _BE,),
        in_specs=[
            pl.BlockSpec((_BE, NBR), lambda i: (i, 0)),
            pl.BlockSpec((NBR, 256), lambda i: (0, 0)),
            pl.BlockSpec((1, 256), lambda i: (0, 0)),
            pl.BlockSpec((256, 128), lambda i: (0, 0)),
            pl.BlockSpec((1, 128), lambda i: (0, 0)),
            pl.BlockSpec((128, AFL), lambda i: (0, 0)),
            pl.BlockSpec((1, AFL), lambda i: (0, 0)),
        ],
        out_specs=[
            pl.BlockSpec((_BE, 2 * AFL), lambda i: (i, 0)),
            pl.BlockSpec((_GA, AFL), lambda i: (i, 0)),
        ],
        out_shape=[
            jax.ShapeDtypeStruct((E, 2 * AFL), jnp.float32),
            jax.ShapeDtypeStruct((N, AFL), jnp.float32),
        ],
    )(nbr_flat, w1, b1.reshape(1, 256), w2, b2.reshape(1, 128), w3,
      b3.reshape(1, AFL))


def _bn_body(a_ref, x_ref):
    a = a_ref[...][:, :AFL]
    m = jnp.mean(a, axis=0, keepdims=True)
    v = jnp.mean((a - m) * (a - m), axis=0, keepdims=True)
    xn = (a - m) * lax.rsqrt(v + 1e-5)
    x_ref[...] = jnp.concatenate([xn, jnp.zeros_like(xn)], axis=1)


def _bn(a):
    return pl.pallas_call(
        _bn_body,
        out_shape=jax.ShapeDtypeStruct((N, 2 * AFL), jnp.float32),
    )(a)


def _comb_body(x_ref, ezs_ref, ezt_ref, o_ref):
    ezt = ezt_ref[...]
    o_ref[...] = (x_ref[...][:, :AFL] + ezs_ref[...]
                  + ezt[0, :N, :AFL] + ezt[1, :N, :AFL])


def _combine(x, ezs, ezt_p):
    return pl.pallas_call(
        _comb_body,
        out_shape=jax.ShapeDtypeStruct((N, AFL), jnp.float32),
    )(x, ezs, ezt_p)


def _msg_body(xi_ref, comb_ref, we1, be1, we2, be2, o_ref):
    xi = xi_ref[...][:, :AFL]
    xj = jnp.broadcast_to(comb_ref[...].reshape(_GA, 1, AFL),
                          (_GA, M, AFL)).reshape(_BE, AFL)
    cat = jnp.concatenate([xi, xj - xi], axis=1)
    h = jnp.maximum(_mm(cat, we1[...]) + be1[...], 0.0)
    msg = _mm(h, we2[...]) + be2[...]
    o_ref[...] = jnp.concatenate([msg, jnp.zeros_like(msg)], axis=1)


def _msg(xi, comb, we1, be1, we2, be2):
    return pl.pallas_call(
        _msg_body,
        grid=(E // _BE,),
        in_specs=[
            pl.BlockSpec((_BE, 2 * AFL), lambda i: (i, 0)),
            pl.BlockSpec((_GA, AFL), lambda i: (i, 0)),
            pl.BlockSpec((2 * AFL, 256), lambda i: (0, 0)),
            pl.BlockSpec((1, 256), lambda i: (0, 0)),
            pl.BlockSpec((256, AFL), lambda i: (0, 0)),
            pl.BlockSpec((1, AFL), lambda i: (0, 0)),
        ],
        out_specs=pl.BlockSpec((_BE, 2 * AFL), lambda i: (i, 0)),
        out_shape=jax.ShapeDtypeStruct((E, 2 * AFL), jnp.float32),
    )(xi, comb, we1, be1.reshape(1, 256), we2, be2.reshape(1, AFL))


def _tail_body(s_ref, cnt_ref, comb_ref, o_ref):
    cnt_p = cnt_ref[...]
    cnt = jnp.maximum(cnt_p[0, :N, :1] + cnt_p[1, :N, :1], 1.0)
    s_p = s_ref[...]
    agg = (s_p[0, :N, :AFL] + s_p[1, :N, :AFL]) / cnt
    m = jnp.mean(agg, axis=0, keepdims=True)
    v = jnp.mean((agg - m) * (agg - m), axis=0, keepdims=True)
    af = (agg - m) * lax.rsqrt(v + 1e-5) + comb_ref[...]
    o_ref[...] = jnp.concatenate([af, jnp.zeros_like(af)], axis=1)


def _tail(s_p, cnt_p, comb):
    return pl.pallas_call(
        _tail_body,
        out_shape=jax.ShapeDtypeStruct((N, 2 * AFL), jnp.float32),
    )(s_p, cnt_p, comb)


def _head_body(g_ref, wfc, bfc, wh1, bh1, wh2, bh2, wo, bo, o_ref):
    pooled = jnp.mean(g_ref[...][:, :, :AFL], axis=1)
    h = _mm(pooled, wfc[...]) + bfc[...]
    h = jnp.maximum(_mm(h, wh1[...]) + bh1[...], 0.0)
    h = jnp.maximum(_mm(h, wh2[...]) + bh2[...], 0.0)
    o_ref[...] = _mm(h, wo[...]) + bo[...]


def _head(gathered, p):
    return pl.pallas_call(
        _head_body,
        out_shape=jax.ShapeDtypeStruct((NC, 1), jnp.float32),
    )(gathered.reshape(NC, NA, 2 * AFL), p['W_fc'], p['b_fc'].reshape(1, 256),
      p['Wh1'], p['bh1'].reshape(1, 128), p['Wh2'], p['bh2'].reshape(1, 64),
      p['Wout'], p['bout'].reshape(1, 1))


@functools.cache
def _mesh():
    return plsc.VectorSubcoreMesh(core_axis_name="c", subcore_axis_name="s",
                                  num_cores=SC_CORES, num_subcores=SC_SUB)


def _zero_acc(z, acc, s):
    off = pl.multiple_of(s * RPS, 8)
    pltpu.sync_copy(z.at[pl.ds(off, RPS)], acc.at[pl.ds(off, RPS)])


def _dump_acc(acc, out, c, s):
    off = pl.multiple_of(s * RPS, 8)
    pltpu.sync_copy(acc.at[pl.ds(off, RPS)], out.at[c, pl.ds(off, RPS)])


def _sc_cnt_body(idx_hbm, z16, ones_hbm, cnt_out,
                 idx_v, ones_v, sems, acc16):
    c = lax.axis_index("c")
    s = lax.axis_index("s")
    w = c * SC_SUB + s
    pltpu.sync_copy(idx_hbm.at[w], idx_v)
    pltpu.sync_copy(ones_hbm, ones_v)
    _zero_acc(z16, acc16, s)
    plsc.subcore_barrier()

    def chunk(j, carry):
        sd = []
        for i in range(GRP):
            row = j * GRP + i
            sd.append(pltpu.async_copy(ones_v, acc16.at[idx_v.at[row]],
                                       sems, add=True))
        for d in sd:
            d.wait()
        return carry

    lax.fori_loop(0, NOUT, chunk, 0)
    plsc.subcore_barrier()
    _dump_acc(acc16, cnt_out, c, s)


def _sc_scatter_gather_body(t_hbm, x_hbm, idx_hbm, z64,
                            ezt_out, xi_out,
                            idx_v, tbuf, xbuf, semt, semg, sems,
                            acc64):
    c = lax.axis_index("c")
    s = lax.axis_index("s")
    w = c * SC_SUB + s
    pltpu.sync_copy(idx_hbm.at[w], idx_v)
    _zero_acc(z64, acc64, s)
    plsc.subcore_barrier()

    base0 = pl.multiple_of(w * EW, 8)
    pltpu.async_copy(t_hbm.at[pl.ds(base0, CH)], tbuf, semt)

    def chunk(j, carry):
        pltpu.make_async_copy(t_hbm.at[pl.ds(0, CH)], tbuf, semt).wait()
        gd = []
        sd = []
        for i in range(GRP):
            row = j * GRP + i
            gd.append(pltpu.async_copy(x_hbm.at[idx_v.at[row]],
                                      xbuf.at[pl.ds(i * IDXW, IDXW)], semg))
            sd.append(pltpu.async_copy(tbuf.at[pl.ds(i * IDXW, IDXW)],
                                       acc64.at[idx_v.at[row]], sems,
                                       add=True))
        for d in sd:
            d.wait()
        nxt = j + 1

        @pl.when(nxt < NOUT)
        def _():
            nb = pl.multiple_of(w * EW + nxt * CH, 8)
            pltpu.async_copy(t_hbm.at[pl.ds(nb, CH)], tbuf, semt)

        for d in gd:
            d.wait()
        base = pl.multiple_of(w * EW + j * CH, 8)
        pltpu.sync_copy(xbuf, xi_out.at[pl.ds(base, CH)])
        return carry

    lax.fori_loop(0, NOUT, chunk, 0)
    plsc.subcore_barrier()
    _dump_acc(acc64, ezt_out, c, s)


def _sc_scatter_body(v_hbm, idx_hbm, z64, s_out,
                     idx_v, vbuf0, vbuf1, semt0, semt1, sems, acc64):
    c = lax.axis_index("c")
    s = lax.axis_index("s")
    w = c * SC_SUB + s
    pltpu.sync_copy(idx_hbm.at[w], idx_v)
    _zero_acc(z64, acc64, s)
    plsc.subcore_barrier()

    vbufs = ((vbuf0, semt0), (vbuf1, semt1))
    base0 = pl.multiple_of(w * EW, 8)
    pltpu.async_copy(v_hbm.at[pl.ds(base0, CH)], vbuf0, semt0)
    pltpu.async_copy(v_hbm.at[pl.ds(base0 + CH, CH)], vbuf1, semt1)

    def do_chunk(j, b, prefetch):
        vb, st = vbufs[b]
        pltpu.make_async_copy(v_hbm.at[pl.ds(0, CH)], vb, st).wait()
        sd = []
        for i in range(GRP):
            row = j * GRP + i
            sd.append(pltpu.async_copy(vb.at[pl.ds(i * IDXW, IDXW)],
                                       acc64.at[idx_v.at[row]], sems,
                                       add=True))
        for d in sd:
            d.wait()
        if prefetch:
            nxt = j + 2

            @pl.when(nxt < NOUT)
            def _():
                nb = pl.multiple_of(w * EW + nxt * CH, 8)
                pltpu.async_copy(v_hbm.at[pl.ds(nb, CH)], vb, st)

    def outer(k, carry):
        do_chunk(2 * k, 0, True)
        do_chunk(2 * k + 1, 1, True)
        return carry

    lax.fori_loop(0, NOUT // 2, outer, 0)
    do_chunk(NOUT - 1, 0, False)
    plsc.subcore_barrier()
    _dump_acc(acc64, s_out, c, s)


def _sc_pool_body(af_hbm, cidx_hbm, g_out, cidx_v, gbuf, sem):
    c = lax.axis_index("c")
    s = lax.axis_index("s")
    w = c * SC_SUB + s
    pltpu.sync_copy(cidx_hbm.at[w], cidx_v)
    for i in range(PROWS):
        pltpu.async_copy(af_hbm.at[cidx_v.at[i]],
                         gbuf.at[pl.ds(i * PIDXW, PIDXW)], sem).wait()
    pltpu.sync_copy(gbuf, g_out.at[pl.ds(pl.multiple_of(w * PW, 8), PW)])


@functools.cache
def _get_sc_cnt():
  return pl.kernel(
    _sc_cnt_body,
    out_type=jax.ShapeDtypeStruct((SC_CORES, NP, 16), jnp.float32),
    mesh=_mesh(),
    compiler_params=pltpu.CompilerParams(use_tc_tiling_on_sc=False),
    scratch_types=[
        pltpu.VMEM((NROW, IDXW), jnp.int32),
        pltpu.VMEM((IDXW, 16), jnp.float32),
        pltpu.SemaphoreType.DMA,
        pltpu.VMEM_SHARED((NP, 16), jnp.float32),
    ],
)

@functools.cache
def _get_sc_scatter_gather():
  return pl.kernel(
    _sc_scatter_gather_body,
    out_type=(
        jax.ShapeDtypeStruct((SC_CORES, NP, 2 * AFL), jnp.float32),
        jax.ShapeDtypeStruct((E, 2 * AFL), jnp.float32),
    ),
    mesh=_mesh(),
    compiler_params=pltpu.CompilerParams(use_tc_tiling_on_sc=False),
    scratch_types=[
        pltpu.VMEM((NROW, IDXW), jnp.int32),
        pltpu.VMEM((CH, 2 * AFL), jnp.float32),
        pltpu.VMEM((CH, 2 * AFL), jnp.float32),
        pltpu.SemaphoreType.DMA,
        pltpu.SemaphoreType.DMA,
        pltpu.SemaphoreType.DMA,
        pltpu.VMEM_SHARED((NP, 2 * AFL), jnp.float32),
    ],
)

@functools.cache
def _get_sc_scatter():
  return pl.kernel(
    _sc_scatter_body,
    out_type=jax.ShapeDtypeStruct((SC_CORES, NP, 2 * AFL), jnp.float32),
    mesh=_mesh(),
    compiler_params=pltpu.CompilerParams(use_tc_tiling_on_sc=False),
    scratch_types=[
        pltpu.VMEM((NROW, IDXW), jnp.int32),
        pltpu.VMEM((CH, 2 * AFL), jnp.float32),
        pltpu.VMEM((CH, 2 * AFL), jnp.float32),
        pltpu.SemaphoreType.DMA,
        pltpu.SemaphoreType.DMA,
        pltpu.SemaphoreType.DMA,
        pltpu.VMEM_SHARED((NP, 2 * AFL), jnp.float32),
    ],
)

@functools.cache
def _get_sc_pool():
  return pl.kernel(
    _sc_pool_body,
    out_type=jax.ShapeDtypeStruct((NC * NA, 2 * AFL), jnp.float32),
    mesh=_mesh(),
    compiler_params=pltpu.CompilerParams(use_tc_tiling_on_sc=False),
    scratch_types=[
        pltpu.VMEM((PROWS, PIDXW), jnp.int32),
        pltpu.VMEM((PW, 2 * AFL), jnp.float32),
        pltpu.SemaphoreType.DMA,
    ],
)


def _sc_cnt(*args):
    return _get_sc_cnt()(*args)


def _sc_scatter_gather(*args):
    return _get_sc_scatter_gather()(*args)


def _sc_scatter(*args):
    return _get_sc_scatter()(*args)


def _sc_pool(*args):
    return _get_sc_pool()(*args)


def kernel(atom_fea, nbr_fea, nbr_fea_idx, crystal_atom_idx, params):
    p = params
    tgt3d = nbr_fea_idx.astype(jnp.int32).reshape(NW, NROW, IDXW)
    cidx3d = crystal_atom_idx.astype(jnp.int32).reshape(NW, PROWS, PIDXW)
    z64 = jnp.zeros((NP, 2 * AFL), jnp.float32)
    z16 = jnp.zeros((NP, 16), jnp.float32)
    ones16 = jnp.ones((IDXW, 16), jnp.float32)
    nbr_flat = nbr_fea.reshape(E, NBR)

    af = _embed(atom_fea, p['W_emb'], p['b_emb'])
    cnt_p = _sc_cnt(tgt3d, z16, ones16)
    for l in range(3):
        t, ezs = _edge_mlp(nbr_flat,
                           p['c%d_W1' % l], p['c%d_b1' % l],
                           p['c%d_W2' % l], p['c%d_b2' % l],
                           p['c%d_W3' % l], p['c%d_b3' % l])
        x = _bn(af)
        ezt_p, xi = _sc_scatter_gather(t, x, tgt3d, z64)
        comb = _combine(x, ezs, ezt_p)
        msg = _msg(xi, comb, p['c%d_We1' % l], p['c%d_be1' % l],
                   p['c%d_We2' % l], p['c%d_be2' % l])
        s_p = _sc_scatter(msg, tgt3d, z64)
        af = _tail(s_p, cnt_p, comb)

    gathered = _sc_pool(af, cidx3d)
    return _head(gathered, p)
